# initial kernel scaffold (unmeasured)
import jax
import jax.numpy as jnp
from jax import lax
from jax.experimental import pallas as pl
from jax.experimental.pallas import tpu as pltpu

N_DEV = 4
B, SQ, DMODEL = 2, 512, 768
HQ_TOT, DH = 32, 64
H_LOC = HQ_TOT // N_DEV
SKV = 512
BLK = 64
NEG = -1e9


def kernel(x, Wq, K_ext, V_ext, Wo):
    def body(x_ref, wq_ref, k_ref, v_ref, wo_ref, out_ref,
             kv_buf, comm_buf,
             scat_send_sems, scat_recv_sems, local_sems,
             ar_send_sems, ar_recv_sems):
        my = lax.axis_index("i")
        right = (my + 1) % N_DEV

        bar = pltpu.get_barrier_semaphore()
        for off in range(1, N_DEV):
            peer = (my + off) % N_DEV
            pl.semaphore_signal(
                bar, inc=1,
                device_id=(peer,), device_id_type=pl.DeviceIdType.MESH,
            )
        pl.semaphore_wait(bar, N_DEV - 1)

        @pl.when(my == 0)
        def _():
            ck = pltpu.make_async_copy(
                k_ref.at[:, :, pl.ds(0, H_LOC), :], kv_buf.at[0],
                local_sems.at[0])
            cv = pltpu.make_async_copy(
                v_ref.at[:, :, pl.ds(0, H_LOC), :], kv_buf.at[1],
                local_sems.at[1])
            ck.start()
            cv.start()
            rdmas = []
            for d in range(1, N_DEV):
                rk = pltpu.make_async_remote_copy(
                    src_ref=k_ref.at[:, :, pl.ds(d * H_LOC, H_LOC), :],
                    dst_ref=kv_buf.at[0],
                    send_sem=scat_send_sems.at[d - 1, 0],
                    recv_sem=scat_recv_sems.at[0],
                    device_id=(d,), device_id_type=pl.DeviceIdType.MESH,
                )
                rv = pltpu.make_async_remote_copy(
                    src_ref=v_ref.at[:, :, pl.ds(d * H_LOC, H_LOC), :],
                    dst_ref=kv_buf.at[1],
                    send_sem=scat_send_sems.at[d - 1, 1],
                    recv_sem=scat_recv_sems.at[1],
                    device_id=(d,), device_id_type=pl.DeviceIdType.MESH,
                )
                rk.start()
                rv.start()
                rdmas.append((rk, rv))
            ck.wait()
            cv.wait()
            for rk, rv in rdmas:
                rk.wait_send()
                rv.wait_send()

        @pl.when(my != 0)
        def _():
            rk = pltpu.make_async_remote_copy(
                src_ref=kv_buf.at[0], dst_ref=kv_buf.at[0],
                send_sem=scat_send_sems.at[0, 0],
                recv_sem=scat_recv_sems.at[0],
                device_id=(0,), device_id_type=pl.DeviceIdType.MESH,
            )
            rv = pltpu.make_async_remote_copy(
                src_ref=kv_buf.at[1], dst_ref=kv_buf.at[1],
                send_sem=scat_send_sems.at[0, 1],
                recv_sem=scat_recv_sems.at[1],
                device_id=(0,), device_id_type=pl.DeviceIdType.MESH,
            )
            rk.wait_recv()
            rv.wait_recv()

        x_bf = x_ref[:].astype(jnp.bfloat16)
        wq_bf = wq_ref[:].astype(jnp.bfloat16)
        wo_bf = wo_ref[:].astype(jnp.bfloat16)

        qb = lax.broadcasted_iota(jnp.int32, (SQ, SKV), 0) // BLK
        kb = lax.broadcasted_iota(jnp.int32, (SQ, SKV), 1) // BLK
        mask = kb <= qb

        for b in range(B):
            q = jnp.dot(x_bf[b], wq_bf,
                        preferred_element_type=jnp.float32)
            q = q.reshape(SQ, H_LOC, DH).astype(jnp.bfloat16)
            ctx_parts = []
            for h in range(H_LOC):
                kh = kv_buf[0, b, :, h, :].astype(jnp.bfloat16)
                vh = kv_buf[1, b, :, h, :].astype(jnp.bfloat16)
                s = lax.dot_general(
                    q[:, h, :], kh, (((1,), (1,)), ((), ())),
                    preferred_element_type=jnp.float32) * 0.125
                s = jnp.where(mask, s, NEG)
                m = jnp.max(s, axis=-1, keepdims=True)
                w = jnp.exp(s - m)
                w = w / jnp.sum(w, axis=-1, keepdims=True)
                ctx_parts.append(
                    jnp.dot(w.astype(jnp.bfloat16), vh,
                            preferred_element_type=jnp.float32))
            ctx = jnp.concatenate(ctx_parts, axis=-1)
            partial = jnp.dot(ctx.astype(jnp.bfloat16), wo_bf,
                              preferred_element_type=jnp.float32)
            out_ref[b] = partial
            comm_buf[0, b] = partial

        for h in range(N_DEV - 1):
            rdma = pltpu.make_async_remote_copy(
                src_ref=comm_buf.at[h],
                dst_ref=comm_buf.at[h + 1],
                send_sem=ar_send_sems.at[h],
                recv_sem=ar_recv_sems.at[h],
                device_id=(right,), device_id_type=pl.DeviceIdType.MESH,
            )
            rdma.start()
            rdma.wait()
            out_ref[:] = out_ref[:] + comm_buf[h + 1]

    out_shape = jax.ShapeDtypeStruct((B, SQ, DMODEL), jnp.float32)
    return pl.pallas_call(
        body,
        out_shape=out_shape,
        in_specs=[
            pl.BlockSpec(memory_space=pltpu.VMEM),
            pl.BlockSpec(memory_space=pltpu.VMEM),
            pl.BlockSpec(memory_space=pltpu.ANY),
            pl.BlockSpec(memory_space=pltpu.ANY),
            pl.BlockSpec(memory_space=pltpu.VMEM),
        ],
        out_specs=pl.BlockSpec(memory_space=pltpu.VMEM),
        scratch_shapes=[
            pltpu.VMEM((2, B, SKV, H_LOC, DH), jnp.float32),
            pltpu.VMEM((N_DEV, B, SQ, DMODEL), jnp.float32),
            pltpu.SemaphoreType.DMA((N_DEV - 1, 2)),
            pltpu.SemaphoreType.DMA((2,)),
            pltpu.SemaphoreType.DMA((2,)),
            pltpu.SemaphoreType.DMA((N_DEV - 1,)),
            pltpu.SemaphoreType.DMA((N_DEV - 1,)),
        ],
        compiler_params=pltpu.CompilerParams(
            collective_id=0,
            vmem_limit_bytes=100 * 1024 * 1024,
        ),
    )(x, Wq, K_ext, V_ext, Wo)


# baseline (device time: 350351 ns/iter reference)
import jax
import jax.numpy as jnp
from jax import lax
from jax.experimental import pallas as pl
from jax.experimental.pallas import tpu as pltpu

N_DEV = 4
B, SQ, DMODEL = 2, 512, 768
HQ_TOT, DH = 32, 64
H_LOC = HQ_TOT // N_DEV
SKV = 512
BLK = 64
NEG = -1e9


def kernel(x, Wq, K_ext, V_ext, Wo):
    def body(x_ref, wq_ref, k_ref, v_ref, wo_ref, out_ref,
             kv_buf, comm_buf,
             scat_send_sems, scat_recv_sems, local_sems,
             ar_send_sems, ar_recv_sems):
        my = lax.axis_index("i")
        right = (my + 1) % N_DEV

        bar = pltpu.get_barrier_semaphore()
        for off in range(1, N_DEV):
            peer = (my + off) % N_DEV
            pl.semaphore_signal(
                bar, inc=1,
                device_id=(peer,), device_id_type=pl.DeviceIdType.MESH,
            )
        pl.semaphore_wait(bar, N_DEV - 1)

        @pl.when(my == 0)
        def _():
            ck = pltpu.make_async_copy(
                k_ref.at[:, :, pl.ds(0, H_LOC), :], kv_buf.at[0],
                local_sems.at[0])
            cv = pltpu.make_async_copy(
                v_ref.at[:, :, pl.ds(0, H_LOC), :], kv_buf.at[1],
                local_sems.at[1])
            ck.start()
            cv.start()
            rdmas = []
            for d in range(1, N_DEV):
                rk = pltpu.make_async_remote_copy(
                    src_ref=k_ref.at[:, :, pl.ds(d * H_LOC, H_LOC), :],
                    dst_ref=kv_buf.at[0],
                    send_sem=scat_send_sems.at[d - 1, 0],
                    recv_sem=scat_recv_sems.at[0],
                    device_id=(d,), device_id_type=pl.DeviceIdType.MESH,
                )
                rv = pltpu.make_async_remote_copy(
                    src_ref=v_ref.at[:, :, pl.ds(d * H_LOC, H_LOC), :],
                    dst_ref=kv_buf.at[1],
                    send_sem=scat_send_sems.at[d - 1, 1],
                    recv_sem=scat_recv_sems.at[1],
                    device_id=(d,), device_id_type=pl.DeviceIdType.MESH,
                )
                rk.start()
                rv.start()
                rdmas.append((rk, rv))
            ck.wait()
            cv.wait()
            for rk, rv in rdmas:
                rk.wait_send()
                rv.wait_send()

        @pl.when(my != 0)
        def _():
            rk = pltpu.make_async_remote_copy(
                src_ref=kv_buf.at[0], dst_ref=kv_buf.at[0],
                send_sem=scat_send_sems.at[0, 0],
                recv_sem=scat_recv_sems.at[0],
                device_id=(0,), device_id_type=pl.DeviceIdType.MESH,
            )
            rv = pltpu.make_async_remote_copy(
                src_ref=kv_buf.at[1], dst_ref=kv_buf.at[1],
                send_sem=scat_send_sems.at[0, 1],
                recv_sem=scat_recv_sems.at[1],
                device_id=(0,), device_id_type=pl.DeviceIdType.MESH,
            )
            rk.wait_recv()
            rv.wait_recv()

        x_bf = x_ref[:].astype(jnp.bfloat16)
        wq_bf = wq_ref[:].astype(jnp.bfloat16)
        wo_bf = wo_ref[:].astype(jnp.bfloat16)

        qb = lax.broadcasted_iota(jnp.int32, (SQ, SKV), 0) // BLK
        kb = lax.broadcasted_iota(jnp.int32, (SQ, SKV), 1) // BLK
        mask = kb <= qb

        for b in range(B):
            q = jnp.dot(x_bf[b], wq_bf,
                        preferred_element_type=jnp.float32)
            q = q.reshape(SQ, H_LOC, DH).astype(jnp.bfloat16)
            ctx_parts = []
            for h in range(H_LOC):
                kh = kv_buf[0, b, :, h, :].astype(jnp.bfloat16)
                vh = kv_buf[1, b, :, h, :].astype(jnp.bfloat16)
                s = lax.dot_general(
                    q[:, h, :], kh, (((1,), (1,)), ((), ())),
                    preferred_element_type=jnp.float32) * 0.125
                s = jnp.where(mask, s, NEG)
                m = jnp.max(s, axis=-1, keepdims=True)
                w = jnp.exp(s - m)
                w = w / jnp.sum(w, axis=-1, keepdims=True)
                ctx_parts.append(
                    jnp.dot(w.astype(jnp.bfloat16), vh,
                            preferred_element_type=jnp.float32))
            ctx = jnp.concatenate(ctx_parts, axis=-1)
            partial = jnp.dot(ctx.astype(jnp.bfloat16), wo_bf,
                              preferred_element_type=jnp.float32)
            out_ref[b] = partial
            comm_buf[0, b] = partial

        for h in range(N_DEV - 1):
            rdma = pltpu.make_async_remote_copy(
                src_ref=comm_buf.at[h],
                dst_ref=comm_buf.at[h + 1],
                send_sem=ar_send_sems.at[h],
                recv_sem=ar_recv_sems.at[h],
                device_id=(right,), device_id_type=pl.DeviceIdType.MESH,
            )
            rdma.start()
            rdma.wait()
            out_ref[:] = out_ref[:] + comm_buf[h + 1]

    out_shape = jax.ShapeDtypeStruct((B, SQ, DMODEL), jnp.float32)
    return pl.pallas_call(
        body,
        out_shape=out_shape,
        in_specs=[
            pl.BlockSpec(memory_space=pltpu.VMEM),
            pl.BlockSpec(memory_space=pltpu.VMEM),
            pl.BlockSpec(memory_space=pl.ANY),
            pl.BlockSpec(memory_space=pl.ANY),
            pl.BlockSpec(memory_space=pltpu.VMEM),
        ],
        out_specs=pl.BlockSpec(memory_space=pltpu.VMEM),
        scratch_shapes=[
            pltpu.VMEM((2, B, SKV, H_LOC, DH), jnp.float32),
            pltpu.VMEM((N_DEV, B, SQ, DMODEL), jnp.float32),
            pltpu.SemaphoreType.DMA((N_DEV - 1, 2)),
            pltpu.SemaphoreType.DMA((2,)),
            pltpu.SemaphoreType.DMA((2,)),
            pltpu.SemaphoreType.DMA((N_DEV - 1,)),
            pltpu.SemaphoreType.DMA((N_DEV - 1,)),
        ],
        compiler_params=pltpu.CompilerParams(
            collective_id=0,
            vmem_limit_bytes=100 * 1024 * 1024,
        ),
    )(x, Wq, K_ext, V_ext, Wo)


# device time: 129686 ns/iter; 2.7015x vs baseline; 2.7015x over previous
import os

import jax
import jax.numpy as jnp
from jax import lax
from jax.experimental import pallas as pl
from jax.experimental.pallas import tpu as pltpu

_SKIP_SCATTER = os.environ.get("KSKIP_SCATTER") == "1"
_SKIP_COMPUTE = os.environ.get("KSKIP_COMPUTE") == "1"
_SKIP_AR = os.environ.get("KSKIP_AR") == "1"

N_DEV = 4
B, SQ, DMODEL = 2, 512, 768
HQ_TOT, DH = 32, 64
H_LOC = HQ_TOT // N_DEV
HF = H_LOC * DH
SKV = 512
BLK = 64
NEG = -1e9
CHK = SQ // N_DEV


def kernel(x, Wq, K_ext, V_ext, Wo):
    K2 = K_ext.reshape(B, SKV, HQ_TOT * DH)
    V2 = V_ext.reshape(B, SKV, HQ_TOT * DH)

    def body(x_ref, wq_ref, k_ref, v_ref, wo_ref, out_ref,
             kvfull, kv_send, kv_buf,
             rs_stage, rs_recv, ag_recv,
             full_sems, scat_send_sems, scat_recv_sem,
             rs_send_sems, rs_recv_sems, ag_send_sems, ag_recv_sems):
        my = lax.axis_index("i")
        right = (my + 1) % N_DEV

        bar = pltpu.get_barrier_semaphore()
        for off in range(1, N_DEV):
            peer = (my + off) % N_DEV
            pl.semaphore_signal(
                bar, inc=1,
                device_id=(peer,), device_id_type=pl.DeviceIdType.MESH,
            )
        pl.semaphore_wait(bar, N_DEV - 1)

        is_src = my == 0

        if not _SKIP_SCATTER:
            @pl.when(is_src)
            def _():
                pltpu.make_async_copy(
                    k_ref, kvfull.at[0], full_sems.at[0]).start()
                pltpu.make_async_copy(
                    v_ref, kvfull.at[1], full_sems.at[1]).start()

        x_bf = x_ref[:].astype(jnp.bfloat16)
        wq_bf = wq_ref[:].astype(jnp.bfloat16)
        wo_bf = wo_ref[:].astype(jnp.bfloat16)
        qs = []
        if not _SKIP_COMPUTE:
            for b in range(B):
                q = jnp.dot(x_bf[b], wq_bf,
                            preferred_element_type=jnp.float32)
                qs.append(q.reshape(SQ, H_LOC, DH).astype(jnp.bfloat16))

        if not _SKIP_SCATTER:
            @pl.when(is_src)
            def _():
                pltpu.make_async_copy(
                    k_ref, kvfull.at[0], full_sems.at[0]).wait()
                pltpu.make_async_copy(
                    v_ref, kvfull.at[1], full_sems.at[1]).wait()
                kv = kvfull[:]
                kv_buf[:] = kv[:, :, :, 0:HF].astype(jnp.bfloat16)
                for d in range(1, N_DEV):
                    kv_send[d - 1] = kv[:, :, :, d * HF:(d + 1) * HF].astype(
                        jnp.bfloat16)
                for d in range(1, N_DEV):
                    pltpu.make_async_remote_copy(
                        src_ref=kv_send.at[d - 1],
                        dst_ref=kv_buf,
                        send_sem=scat_send_sems.at[d - 1],
                        recv_sem=scat_recv_sem,
                        device_id=(d,), device_id_type=pl.DeviceIdType.MESH,
                    ).start()

            @pl.when(jnp.logical_not(is_src))
            def _():
                pltpu.make_async_remote_copy(
                    src_ref=kv_buf, dst_ref=kv_buf,
                    send_sem=scat_send_sems.at[0],
                    recv_sem=scat_recv_sem,
                    device_id=(0,), device_id_type=pl.DeviceIdType.MESH,
                ).wait_recv()

        if not _SKIP_COMPUTE:
            qb = lax.broadcasted_iota(jnp.int32, (SQ, SKV), 0) // BLK
            kb = lax.broadcasted_iota(jnp.int32, (SQ, SKV), 1) // BLK
            mask = kb <= qb
            for b in range(B):
                ctx_parts = []
                for h in range(H_LOC):
                    kh = kv_buf[0, b, :, h * DH:(h + 1) * DH]
                    vh = kv_buf[1, b, :, h * DH:(h + 1) * DH]
                    s = lax.dot_general(
                        qs[b][:, h, :], kh, (((1,), (1,)), ((), ())),
                        preferred_element_type=jnp.float32) * 0.125
                    s = jnp.where(mask, s, NEG)
                    m = jnp.max(s, axis=-1, keepdims=True)
                    w = jnp.exp(s - m)
                    w = w / jnp.sum(w, axis=-1, keepdims=True)
                    ctx_parts.append(
                        jnp.dot(w.astype(jnp.bfloat16), vh,
                                preferred_element_type=jnp.float32))
                ctx = jnp.concatenate(ctx_parts, axis=-1)
                out_ref[b] = jnp.dot(ctx.astype(jnp.bfloat16), wo_bf,
                                     preferred_element_type=jnp.float32)
        else:
            out_ref[:] = jnp.zeros((B, SQ, DMODEL), jnp.float32)

        if not _SKIP_SCATTER:
            @pl.when(is_src)
            def _():
                for d in range(1, N_DEV):
                    pltpu.make_async_remote_copy(
                        src_ref=kv_send.at[d - 1], dst_ref=kv_buf,
                        send_sem=scat_send_sems.at[d - 1],
                        recv_sem=scat_recv_sem,
                        device_id=(d,), device_id_type=pl.DeviceIdType.MESH,
                    ).wait_send()

        if not _SKIP_AR:
            for s in range(N_DEV - 1):
                send_idx = (my - s) % N_DEV
                recv_idx = (my - s - 1) % N_DEV
                rs_stage[:] = out_ref[
                    :, pl.ds(send_idx * CHK, CHK), :].astype(jnp.bfloat16)
                rdma = pltpu.make_async_remote_copy(
                    src_ref=rs_stage, dst_ref=rs_recv.at[s],
                    send_sem=rs_send_sems.at[s], recv_sem=rs_recv_sems.at[s],
                    device_id=(right,), device_id_type=pl.DeviceIdType.MESH,
                )
                rdma.start()
                rdma.wait()
                out_ref[:, pl.ds(recv_idx * CHK, CHK), :] = (
                    out_ref[:, pl.ds(recv_idx * CHK, CHK), :]
                    + rs_recv[s].astype(jnp.float32))

            my_red = (my + 1) % N_DEV
            rs_stage[:] = out_ref[
                :, pl.ds(my_red * CHK, CHK), :].astype(jnp.bfloat16)
            for s in range(N_DEV - 1):
                src = rs_stage if s == 0 else ag_recv.at[s - 1]
                rdma = pltpu.make_async_remote_copy(
                    src_ref=src, dst_ref=ag_recv.at[s],
                    send_sem=ag_send_sems.at[s], recv_sem=ag_recv_sems.at[s],
                    device_id=(right,), device_id_type=pl.DeviceIdType.MESH,
                )
                rdma.start()
                rdma.wait()
                idx = (my - s) % N_DEV
                out_ref[:, pl.ds(idx * CHK, CHK), :] = (
                    ag_recv[s].astype(jnp.float32))

    out_shape = jax.ShapeDtypeStruct((B, SQ, DMODEL), jnp.float32)
    return pl.pallas_call(
        body,
        out_shape=out_shape,
        in_specs=[
            pl.BlockSpec(memory_space=pltpu.VMEM),
            pl.BlockSpec(memory_space=pltpu.VMEM),
            pl.BlockSpec(memory_space=pl.ANY),
            pl.BlockSpec(memory_space=pl.ANY),
            pl.BlockSpec(memory_space=pltpu.VMEM),
        ],
        out_specs=pl.BlockSpec(memory_space=pltpu.VMEM),
        scratch_shapes=[
            pltpu.VMEM((2, B, SKV, HQ_TOT * DH), jnp.float32),
            pltpu.VMEM((N_DEV - 1, 2, B, SKV, HF), jnp.bfloat16),
            pltpu.VMEM((2, B, SKV, HF), jnp.bfloat16),
            pltpu.VMEM((B, CHK, DMODEL), jnp.bfloat16),
            pltpu.VMEM((N_DEV - 1, B, CHK, DMODEL), jnp.bfloat16),
            pltpu.VMEM((N_DEV - 1, B, CHK, DMODEL), jnp.bfloat16),
            pltpu.SemaphoreType.DMA((2,)),
            pltpu.SemaphoreType.DMA((N_DEV - 1,)),
            pltpu.SemaphoreType.DMA,
            pltpu.SemaphoreType.DMA((N_DEV - 1,)),
            pltpu.SemaphoreType.DMA((N_DEV - 1,)),
            pltpu.SemaphoreType.DMA((N_DEV - 1,)),
            pltpu.SemaphoreType.DMA((N_DEV - 1,)),
        ],
        compiler_params=pltpu.CompilerParams(
            collective_id=0,
            vmem_limit_bytes=120 * 1024 * 1024,
        ),
    )(x, Wq, K2, V2, Wo)


# device time: 109161 ns/iter; 3.2095x vs baseline; 1.1880x over previous
import jax
import jax.numpy as jnp
from jax import lax
from jax.experimental import pallas as pl
from jax.experimental.pallas import tpu as pltpu

N_DEV = 4
B, SQ, DMODEL = 2, 512, 768
HQ_TOT, DH = 32, 64
H_LOC = HQ_TOT // N_DEV
HF = H_LOC * DH
SKV = 512
HALF = SKV // 2
BLK = 64
NEG = -1e9
CHK = SQ // N_DEV


def kernel(x, Wq, K_ext, V_ext, Wo):
    K2 = K_ext.reshape(B, SKV, HQ_TOT * DH)
    V2 = V_ext.reshape(B, SKV, HQ_TOT * DH)

    def body(x_ref, wq_ref, k_ref, v_ref, wo_ref, out_ref,
             kvfull, kv_send, kv_buf,
             rs_stage, rs_recv, ag_stage, ag_recv,
             full_sems, scat_send_sems, scat_recv_sems,
             rs_send_sems, rs_recv_sems, ag_send_sems, ag_recv_sems):
        my = lax.axis_index("i")
        is_src = my == 0
        own = (my + 1) % N_DEV

        bar = pltpu.get_barrier_semaphore()
        for off in range(1, N_DEV):
            peer = (my + off) % N_DEV
            pl.semaphore_signal(
                bar, inc=1,
                device_id=(peer,), device_id_type=pl.DeviceIdType.MESH,
            )
        pl.semaphore_wait(bar, N_DEV - 1)

        @pl.when(is_src)
        def _():
            for hf in range(2):
                rows = pl.ds(hf * HALF, HALF)
                pltpu.make_async_copy(
                    k_ref.at[:, rows, :], kvfull.at[0, :, rows, :],
                    full_sems.at[hf, 0]).start()
                pltpu.make_async_copy(
                    v_ref.at[:, rows, :], kvfull.at[1, :, rows, :],
                    full_sems.at[hf, 1]).start()

        x_bf = x_ref[:].astype(jnp.bfloat16)
        wq_bf = wq_ref[:].astype(jnp.bfloat16)
        wo_bf = wo_ref[:].astype(jnp.bfloat16)
        qs = []
        for b in range(B):
            q = jnp.dot(x_bf[b], wq_bf,
                        preferred_element_type=jnp.float32)
            qs.append(q.reshape(SQ, H_LOC, DH).astype(jnp.bfloat16))

        @pl.when(is_src)
        def _():
            for hf in range(2):
                rows = pl.ds(hf * HALF, HALF)
                pltpu.make_async_copy(
                    k_ref.at[:, rows, :], kvfull.at[0, :, rows, :],
                    full_sems.at[hf, 0]).wait()
                pltpu.make_async_copy(
                    v_ref.at[:, rows, :], kvfull.at[1, :, rows, :],
                    full_sems.at[hf, 1]).wait()
                kvh = kvfull[:, :, hf * HALF:(hf + 1) * HALF, :]
                kv_buf[:, :, rows, :] = kvh[:, :, :, 0:HF].astype(
                    jnp.bfloat16)
                for d in range(1, N_DEV):
                    kv_send[hf, d - 1] = kvh[
                        :, :, :, d * HF:(d + 1) * HF].astype(jnp.bfloat16)
                for d in range(1, N_DEV):
                    pltpu.make_async_remote_copy(
                        src_ref=kv_send.at[hf, d - 1],
                        dst_ref=kv_buf.at[:, :, rows, :],
                        send_sem=scat_send_sems.at[hf, d - 1],
                        recv_sem=scat_recv_sems.at[hf],
                        device_id=(d,), device_id_type=pl.DeviceIdType.MESH,
                    ).start()

        def wait_half(hf):
            rows = pl.ds(hf * HALF, HALF)
            pltpu.make_async_remote_copy(
                src_ref=kv_buf.at[:, :, rows, :],
                dst_ref=kv_buf.at[:, :, rows, :],
                send_sem=scat_send_sems.at[hf, 0],
                recv_sem=scat_recv_sems.at[hf],
                device_id=(0,), device_id_type=pl.DeviceIdType.MESH,
            ).wait_recv()

        qb_base = lax.broadcasted_iota(jnp.int32, (CHK, SKV), 0)
        kb_full = lax.broadcasted_iota(jnp.int32, (CHK, SKV), 1) // BLK

        for c in range(N_DEV):
            if c == 0:
                @pl.when(jnp.logical_not(is_src))
                def _():
                    wait_half(0)
            if c == 2:
                @pl.when(jnp.logical_not(is_src))
                def _():
                    wait_half(1)
            kl = CHK * (c + 1)
            mask = kb_full[:, :kl] <= (qb_base[:, :kl] + c * CHK) // BLK
            for b in range(B):
                ctx_parts = []
                for h in range(H_LOC):
                    kh = kv_buf[0, b, 0:kl, h * DH:(h + 1) * DH]
                    vh = kv_buf[1, b, 0:kl, h * DH:(h + 1) * DH]
                    qch = qs[b][c * CHK:(c + 1) * CHK, h, :]
                    s = lax.dot_general(
                        qch, kh, (((1,), (1,)), ((), ())),
                        preferred_element_type=jnp.float32) * 0.125
                    s = jnp.where(mask, s, NEG)
                    m = jnp.max(s, axis=-1, keepdims=True)
                    w = jnp.exp(s - m)
                    w = w / jnp.sum(w, axis=-1, keepdims=True)
                    ctx_parts.append(
                        jnp.dot(w.astype(jnp.bfloat16), vh,
                                preferred_element_type=jnp.float32))
                ctx = jnp.concatenate(ctx_parts, axis=-1)
                part = jnp.dot(ctx.astype(jnp.bfloat16), wo_bf,
                               preferred_element_type=jnp.float32)
                out_ref[b, c * CHK:(c + 1) * CHK, :] = part
                rs_stage[c, b] = part.astype(jnp.bfloat16)
            p = (c - 1) % N_DEV
            @pl.when(jnp.int32(c) != own)
            def _():
                slot = (my - p) % N_DEV - 1
                pltpu.make_async_remote_copy(
                    src_ref=rs_stage.at[c],
                    dst_ref=rs_recv.at[slot],
                    send_sem=rs_send_sems.at[c],
                    recv_sem=rs_recv_sems.at[slot],
                    device_id=(p,), device_id_type=pl.DeviceIdType.MESH,
                ).start()

        for slot in range(N_DEV - 1):
            pltpu.make_async_remote_copy(
                src_ref=rs_recv.at[slot], dst_ref=rs_recv.at[slot],
                send_sem=rs_send_sems.at[0],
                recv_sem=rs_recv_sems.at[slot],
                device_id=(0,), device_id_type=pl.DeviceIdType.MESH,
            ).wait_recv()
        red = out_ref[:, pl.ds(own * CHK, CHK), :]
        for slot in range(N_DEV - 1):
            red = red + rs_recv[slot].astype(jnp.float32)
        out_ref[:, pl.ds(own * CHK, CHK), :] = red
        ag_stage[:] = red.astype(jnp.bfloat16)

        for off in range(1, N_DEV):
            p = (my + off) % N_DEV
            pltpu.make_async_remote_copy(
                src_ref=ag_stage,
                dst_ref=ag_recv.at[N_DEV - 1 - off],
                send_sem=ag_send_sems.at[off - 1],
                recv_sem=ag_recv_sems.at[N_DEV - 1 - off],
                device_id=(p,), device_id_type=pl.DeviceIdType.MESH,
            ).start()
        for off in range(1, N_DEV):
            slot = off - 1
            pltpu.make_async_remote_copy(
                src_ref=ag_recv.at[slot], dst_ref=ag_recv.at[slot],
                send_sem=ag_send_sems.at[0],
                recv_sem=ag_recv_sems.at[slot],
                device_id=(0,), device_id_type=pl.DeviceIdType.MESH,
            ).wait_recv()
            src_chunk = (my + off + 1) % N_DEV
            out_ref[:, pl.ds(src_chunk * CHK, CHK), :] = (
                ag_recv[slot].astype(jnp.float32))

        @pl.when(is_src)
        def _():
            for hf in range(2):
                for d in range(1, N_DEV):
                    pltpu.make_async_remote_copy(
                        src_ref=kv_send.at[hf, d - 1],
                        dst_ref=kv_buf.at[:, :, pl.ds(hf * HALF, HALF), :],
                        send_sem=scat_send_sems.at[hf, d - 1],
                        recv_sem=scat_recv_sems.at[hf],
                        device_id=(d,), device_id_type=pl.DeviceIdType.MESH,
                    ).wait_send()
        for c in range(N_DEV):
            @pl.when(jnp.int32(c) != own)
            def _():
                pltpu.make_async_remote_copy(
                    src_ref=rs_stage.at[c], dst_ref=rs_recv.at[0],
                    send_sem=rs_send_sems.at[c],
                    recv_sem=rs_recv_sems.at[0],
                    device_id=(0,), device_id_type=pl.DeviceIdType.MESH,
                ).wait_send()
        for off in range(1, N_DEV):
            pltpu.make_async_remote_copy(
                src_ref=ag_stage, dst_ref=ag_recv.at[0],
                send_sem=ag_send_sems.at[off - 1],
                recv_sem=ag_recv_sems.at[0],
                device_id=(0,), device_id_type=pl.DeviceIdType.MESH,
            ).wait_send()

    out_shape = jax.ShapeDtypeStruct((B, SQ, DMODEL), jnp.float32)
    return pl.pallas_call(
        body,
        out_shape=out_shape,
        in_specs=[
            pl.BlockSpec(memory_space=pltpu.VMEM),
            pl.BlockSpec(memory_space=pltpu.VMEM),
            pl.BlockSpec(memory_space=pl.ANY),
            pl.BlockSpec(memory_space=pl.ANY),
            pl.BlockSpec(memory_space=pltpu.VMEM),
        ],
        out_specs=pl.BlockSpec(memory_space=pltpu.VMEM),
        scratch_shapes=[
            pltpu.VMEM((2, B, SKV, HQ_TOT * DH), jnp.float32),
            pltpu.VMEM((2, N_DEV - 1, 2, B, HALF, HF), jnp.bfloat16),
            pltpu.VMEM((2, B, SKV, HF), jnp.bfloat16),
            pltpu.VMEM((N_DEV, B, CHK, DMODEL), jnp.bfloat16),
            pltpu.VMEM((N_DEV - 1, B, CHK, DMODEL), jnp.bfloat16),
            pltpu.VMEM((B, CHK, DMODEL), jnp.bfloat16),
            pltpu.VMEM((N_DEV - 1, B, CHK, DMODEL), jnp.bfloat16),
            pltpu.SemaphoreType.DMA((2, 2)),
            pltpu.SemaphoreType.DMA((2, N_DEV - 1)),
            pltpu.SemaphoreType.DMA((2,)),
            pltpu.SemaphoreType.DMA((N_DEV,)),
            pltpu.SemaphoreType.DMA((N_DEV - 1,)),
            pltpu.SemaphoreType.DMA((N_DEV - 1,)),
            pltpu.SemaphoreType.DMA((N_DEV - 1,)),
        ],
        compiler_params=pltpu.CompilerParams(
            collective_id=0,
            vmem_limit_bytes=120 * 1024 * 1024,
        ),
    )(x, Wq, K2, V2, Wo)


# device time: 105331 ns/iter; 3.3262x vs baseline; 1.0364x over previous
import jax
import jax.numpy as jnp
from jax import lax
from jax.experimental import pallas as pl
from jax.experimental.pallas import tpu as pltpu

N_DEV = 4
B, SQ, DMODEL = 2, 512, 768
HQ_TOT, DH = 32, 64
H_LOC = HQ_TOT // N_DEV
HF = H_LOC * DH
SKV = 512
HALF = SKV // 2
BLK = 64
NEG = -1e9
CHK = SQ // N_DEV


def kernel(x, Wq, K_ext, V_ext, Wo):
    K2 = K_ext.reshape(B, SKV, HQ_TOT * DH).astype(jnp.bfloat16)
    V2 = V_ext.reshape(B, SKV, HQ_TOT * DH).astype(jnp.bfloat16)

    def body(x_ref, wq_ref, k_ref, v_ref, wo_ref, out_ref,
             kvfull, kv_send, kv_buf,
             rs_stage, rs_recv, ag_stage, ag_recv,
             full_sems, scat_send_sems, scat_recv_sems,
             rs_send_sems, rs_recv_sems, ag_send_sems, ag_recv_sems):
        my = lax.axis_index("i")
        is_src = my == 0
        own = (my + 1) % N_DEV

        bar = pltpu.get_barrier_semaphore()
        for off in range(1, N_DEV):
            peer = (my + off) % N_DEV
            pl.semaphore_signal(
                bar, inc=1,
                device_id=(peer,), device_id_type=pl.DeviceIdType.MESH,
            )
        pl.semaphore_wait(bar, N_DEV - 1)

        @pl.when(is_src)
        def _():
            for hf in range(2):
                rows = pl.ds(hf * HALF, HALF)
                pltpu.make_async_copy(
                    k_ref.at[:, rows, :], kvfull.at[0, :, rows, :],
                    full_sems.at[hf, 0]).start()
                pltpu.make_async_copy(
                    v_ref.at[:, rows, :], kvfull.at[1, :, rows, :],
                    full_sems.at[hf, 1]).start()

        x_bf = x_ref[:].astype(jnp.bfloat16)
        wq_bf = wq_ref[:].astype(jnp.bfloat16)
        wo_bf = wo_ref[:].astype(jnp.bfloat16)
        qs = []
        for b in range(B):
            q = jnp.dot(x_bf[b], wq_bf,
                        preferred_element_type=jnp.float32)
            qs.append(q.reshape(SQ, H_LOC, DH).astype(jnp.bfloat16))

        @pl.when(is_src)
        def _():
            for hf in range(2):
                rows = pl.ds(hf * HALF, HALF)
                pltpu.make_async_copy(
                    k_ref.at[:, rows, :], kvfull.at[0, :, rows, :],
                    full_sems.at[hf, 0]).wait()
                pltpu.make_async_copy(
                    v_ref.at[:, rows, :], kvfull.at[1, :, rows, :],
                    full_sems.at[hf, 1]).wait()
                kvh = kvfull[:, :, hf * HALF:(hf + 1) * HALF, :]
                for d in range(1, N_DEV):
                    kv_send[hf, d - 1] = kvh[:, :, :, d * HF:(d + 1) * HF]
                for d in range(1, N_DEV):
                    pltpu.make_async_remote_copy(
                        src_ref=kv_send.at[hf, d - 1],
                        dst_ref=kv_buf.at[:, :, rows, :],
                        send_sem=scat_send_sems.at[hf, d - 1],
                        recv_sem=scat_recv_sems.at[hf],
                        device_id=(d,), device_id_type=pl.DeviceIdType.MESH,
                    ).start()
                kv_buf[:, :, rows, :] = kvh[:, :, :, 0:HF]

        def wait_half(hf):
            rows = pl.ds(hf * HALF, HALF)
            pltpu.make_async_remote_copy(
                src_ref=kv_buf.at[:, :, rows, :],
                dst_ref=kv_buf.at[:, :, rows, :],
                send_sem=scat_send_sems.at[hf, 0],
                recv_sem=scat_recv_sems.at[hf],
                device_id=(0,), device_id_type=pl.DeviceIdType.MESH,
            ).wait_recv()

        qb_base = lax.broadcasted_iota(jnp.int32, (CHK, SKV), 0)
        kb_full = lax.broadcasted_iota(jnp.int32, (CHK, SKV), 1) // BLK

        for c in range(N_DEV):
            if c == 0:
                @pl.when(jnp.logical_not(is_src))
                def _():
                    wait_half(0)
            if c == 2:
                @pl.when(jnp.logical_not(is_src))
                def _():
                    wait_half(1)
            kl = CHK * (c + 1)
            mask = kb_full[:, :kl] <= (qb_base[:, :kl] + c * CHK) // BLK
            for b in range(B):
                ctx_parts = []
                for h in range(H_LOC):
                    kh = kv_buf[0, b, 0:kl, h * DH:(h + 1) * DH]
                    vh = kv_buf[1, b, 0:kl, h * DH:(h + 1) * DH]
                    qch = qs[b][c * CHK:(c + 1) * CHK, h, :]
                    s = lax.dot_general(
                        qch, kh, (((1,), (1,)), ((), ())),
                        preferred_element_type=jnp.float32) * 0.125
                    s = jnp.where(mask, s, NEG)
                    m = jnp.max(s, axis=-1, keepdims=True)
                    w = jnp.exp(s - m)
                    w = w / jnp.sum(w, axis=-1, keepdims=True)
                    ctx_parts.append(
                        jnp.dot(w.astype(jnp.bfloat16), vh,
                                preferred_element_type=jnp.float32))
                ctx = jnp.concatenate(ctx_parts, axis=-1)
                part = jnp.dot(ctx.astype(jnp.bfloat16), wo_bf,
                               preferred_element_type=jnp.float32)
                out_ref[b, c * CHK:(c + 1) * CHK, :] = part
                rs_stage[c, b] = part.astype(jnp.bfloat16)
            p = (c - 1) % N_DEV
            @pl.when(jnp.int32(c) != own)
            def _():
                slot = (my - p) % N_DEV - 1
                pltpu.make_async_remote_copy(
                    src_ref=rs_stage.at[c],
                    dst_ref=rs_recv.at[slot],
                    send_sem=rs_send_sems.at[c],
                    recv_sem=rs_recv_sems.at[slot],
                    device_id=(p,), device_id_type=pl.DeviceIdType.MESH,
                ).start()

        for slot in range(N_DEV - 1):
            pltpu.make_async_remote_copy(
                src_ref=rs_recv.at[slot], dst_ref=rs_recv.at[slot],
                send_sem=rs_send_sems.at[0],
                recv_sem=rs_recv_sems.at[slot],
                device_id=(0,), device_id_type=pl.DeviceIdType.MESH,
            ).wait_recv()
        red = out_ref[:, pl.ds(own * CHK, CHK), :]
        for slot in range(N_DEV - 1):
            red = red + rs_recv[slot].astype(jnp.float32)
        out_ref[:, pl.ds(own * CHK, CHK), :] = red
        ag_stage[:] = red.astype(jnp.bfloat16)

        for off in range(1, N_DEV):
            p = (my + off) % N_DEV
            pltpu.make_async_remote_copy(
                src_ref=ag_stage,
                dst_ref=ag_recv.at[N_DEV - 1 - off],
                send_sem=ag_send_sems.at[off - 1],
                recv_sem=ag_recv_sems.at[N_DEV - 1 - off],
                device_id=(p,), device_id_type=pl.DeviceIdType.MESH,
            ).start()
        for off in range(1, N_DEV):
            slot = off - 1
            pltpu.make_async_remote_copy(
                src_ref=ag_recv.at[slot], dst_ref=ag_recv.at[slot],
                send_sem=ag_send_sems.at[0],
                recv_sem=ag_recv_sems.at[slot],
                device_id=(0,), device_id_type=pl.DeviceIdType.MESH,
            ).wait_recv()
            src_chunk = (my + off + 1) % N_DEV
            out_ref[:, pl.ds(src_chunk * CHK, CHK), :] = (
                ag_recv[slot].astype(jnp.float32))

        @pl.when(is_src)
        def _():
            for hf in range(2):
                for d in range(1, N_DEV):
                    pltpu.make_async_remote_copy(
                        src_ref=kv_send.at[hf, d - 1],
                        dst_ref=kv_buf.at[:, :, pl.ds(hf * HALF, HALF), :],
                        send_sem=scat_send_sems.at[hf, d - 1],
                        recv_sem=scat_recv_sems.at[hf],
                        device_id=(d,), device_id_type=pl.DeviceIdType.MESH,
                    ).wait_send()
        for c in range(N_DEV):
            @pl.when(jnp.int32(c) != own)
            def _():
                pltpu.make_async_remote_copy(
                    src_ref=rs_stage.at[c], dst_ref=rs_recv.at[0],
                    send_sem=rs_send_sems.at[c],
                    recv_sem=rs_recv_sems.at[0],
                    device_id=(0,), device_id_type=pl.DeviceIdType.MESH,
                ).wait_send()
        for off in range(1, N_DEV):
            pltpu.make_async_remote_copy(
                src_ref=ag_stage, dst_ref=ag_recv.at[0],
                send_sem=ag_send_sems.at[off - 1],
                recv_sem=ag_recv_sems.at[0],
                device_id=(0,), device_id_type=pl.DeviceIdType.MESH,
            ).wait_send()

    out_shape = jax.ShapeDtypeStruct((B, SQ, DMODEL), jnp.float32)
    return pl.pallas_call(
        body,
        out_shape=out_shape,
        in_specs=[
            pl.BlockSpec(memory_space=pltpu.VMEM),
            pl.BlockSpec(memory_space=pltpu.VMEM),
            pl.BlockSpec(memory_space=pl.ANY),
            pl.BlockSpec(memory_space=pl.ANY),
            pl.BlockSpec(memory_space=pltpu.VMEM),
        ],
        out_specs=pl.BlockSpec(memory_space=pltpu.VMEM),
        scratch_shapes=[
            pltpu.VMEM((2, B, SKV, HQ_TOT * DH), jnp.bfloat16),
            pltpu.VMEM((2, N_DEV - 1, 2, B, HALF, HF), jnp.bfloat16),
            pltpu.VMEM((2, B, SKV, HF), jnp.bfloat16),
            pltpu.VMEM((N_DEV, B, CHK, DMODEL), jnp.bfloat16),
            pltpu.VMEM((N_DEV - 1, B, CHK, DMODEL), jnp.bfloat16),
            pltpu.VMEM((B, CHK, DMODEL), jnp.bfloat16),
            pltpu.VMEM((N_DEV - 1, B, CHK, DMODEL), jnp.bfloat16),
            pltpu.SemaphoreType.DMA((2, 2)),
            pltpu.SemaphoreType.DMA((2, N_DEV - 1)),
            pltpu.SemaphoreType.DMA((2,)),
            pltpu.SemaphoreType.DMA((N_DEV,)),
            pltpu.SemaphoreType.DMA((N_DEV - 1,)),
            pltpu.SemaphoreType.DMA((N_DEV - 1,)),
            pltpu.SemaphoreType.DMA((N_DEV - 1,)),
        ],
        compiler_params=pltpu.CompilerParams(
            collective_id=0,
            vmem_limit_bytes=120 * 1024 * 1024,
        ),
    )(x, Wq, K2, V2, Wo)


# device time: 104668 ns/iter; 3.3473x vs baseline; 1.0063x over previous
import jax
import jax.numpy as jnp
from jax import lax
from jax.experimental import pallas as pl
from jax.experimental.pallas import tpu as pltpu

N_DEV = 4
B, SQ, DMODEL = 2, 512, 768
HQ_TOT, DH = 32, 64
H_LOC = HQ_TOT // N_DEV
HF = H_LOC * DH
SKV = 512
HALF = SKV // 2
BLK = 64
NEG = -1e9
CHK = SQ // N_DEV


def kernel(x, Wq, K_ext, V_ext, Wo):
    K2 = K_ext.reshape(B, SKV, HQ_TOT * DH).astype(jnp.bfloat16)
    V2 = V_ext.reshape(B, SKV, HQ_TOT * DH).astype(jnp.bfloat16)
    xb = x.astype(jnp.bfloat16)
    Wqb = Wq.astype(jnp.bfloat16)
    Wob = Wo.astype(jnp.bfloat16)

    def body(x_ref, wq_ref, k_ref, v_ref, wo_ref, out_ref,
             kvfull, kv_send, kv_buf,
             rs_stage, rs_recv, ag_stage, ag_recv,
             full_sems, scat_send_sems, scat_recv_sems,
             rs_send_sems, rs_recv_sems, ag_send_sems, ag_recv_sems):
        my = lax.axis_index("i")
        is_src = my == 0
        own = (my + 1) % N_DEV

        bar = pltpu.get_barrier_semaphore()
        for off in range(1, N_DEV):
            peer = (my + off) % N_DEV
            pl.semaphore_signal(
                bar, inc=1,
                device_id=(peer,), device_id_type=pl.DeviceIdType.MESH,
            )

        @pl.when(is_src)
        def _():
            for hf in range(2):
                rows = pl.ds(hf * HALF, HALF)
                pltpu.make_async_copy(
                    k_ref.at[:, rows, :], kvfull.at[0, :, rows, :],
                    full_sems.at[hf, 0]).start()
                pltpu.make_async_copy(
                    v_ref.at[:, rows, :], kvfull.at[1, :, rows, :],
                    full_sems.at[hf, 1]).start()

        wo_bf = wo_ref[:]
        qs = []
        for b in range(B):
            q = jnp.dot(x_ref[b], wq_ref[:],
                        preferred_element_type=jnp.float32)
            qs.append(q.reshape(SQ, H_LOC, DH).astype(jnp.bfloat16))

        pl.semaphore_wait(bar, N_DEV - 1)

        @pl.when(is_src)
        def _():
            for hf in range(2):
                rows = pl.ds(hf * HALF, HALF)
                pltpu.make_async_copy(
                    k_ref.at[:, rows, :], kvfull.at[0, :, rows, :],
                    full_sems.at[hf, 0]).wait()
                pltpu.make_async_copy(
                    v_ref.at[:, rows, :], kvfull.at[1, :, rows, :],
                    full_sems.at[hf, 1]).wait()
                kvh = kvfull[:, :, hf * HALF:(hf + 1) * HALF, :]
                for d in range(1, N_DEV):
                    kv_send[hf, d - 1] = kvh[:, :, :, d * HF:(d + 1) * HF]
                for d in range(1, N_DEV):
                    pltpu.make_async_remote_copy(
                        src_ref=kv_send.at[hf, d - 1],
                        dst_ref=kv_buf.at[:, :, rows, :],
                        send_sem=scat_send_sems.at[hf, d - 1],
                        recv_sem=scat_recv_sems.at[hf],
                        device_id=(d,), device_id_type=pl.DeviceIdType.MESH,
                    ).start()
                kv_buf[:, :, rows, :] = kvh[:, :, :, 0:HF]

        def wait_half(hf):
            rows = pl.ds(hf * HALF, HALF)
            pltpu.make_async_remote_copy(
                src_ref=kv_buf.at[:, :, rows, :],
                dst_ref=kv_buf.at[:, :, rows, :],
                send_sem=scat_send_sems.at[hf, 0],
                recv_sem=scat_recv_sems.at[hf],
                device_id=(0,), device_id_type=pl.DeviceIdType.MESH,
            ).wait_recv()

        qb_base = lax.broadcasted_iota(jnp.int32, (CHK, SKV), 0)
        kb_full = lax.broadcasted_iota(jnp.int32, (CHK, SKV), 1) // BLK

        for c in range(N_DEV):
            if c == 0:
                @pl.when(jnp.logical_not(is_src))
                def _():
                    wait_half(0)
            if c == 2:
                @pl.when(jnp.logical_not(is_src))
                def _():
                    wait_half(1)
            kl = CHK * (c + 1)
            mask = kb_full[:, :kl] <= (qb_base[:, :kl] + c * CHK) // BLK
            for b in range(B):
                ctx_parts = []
                for h in range(H_LOC):
                    kh = kv_buf[0, b, 0:kl, h * DH:(h + 1) * DH]
                    vh = kv_buf[1, b, 0:kl, h * DH:(h + 1) * DH]
                    qch = qs[b][c * CHK:(c + 1) * CHK, h, :]
                    s = lax.dot_general(
                        qch, kh, (((1,), (1,)), ((), ())),
                        preferred_element_type=jnp.float32) * 0.125
                    s = jnp.where(mask, s, NEG)
                    m = jnp.max(s, axis=-1, keepdims=True)
                    w = jnp.exp(s - m)
                    w = w / jnp.sum(w, axis=-1, keepdims=True)
                    ctx_parts.append(
                        jnp.dot(w.astype(jnp.bfloat16), vh,
                                preferred_element_type=jnp.float32))
                ctx = jnp.concatenate(ctx_parts, axis=-1)
                part = jnp.dot(ctx.astype(jnp.bfloat16), wo_bf,
                               preferred_element_type=jnp.float32)
                out_ref[b, c * CHK:(c + 1) * CHK, :] = part
                rs_stage[c, b] = part.astype(jnp.bfloat16)
            p = (c - 1) % N_DEV
            @pl.when(jnp.int32(c) != own)
            def _():
                slot = (my - p) % N_DEV - 1
                pltpu.make_async_remote_copy(
                    src_ref=rs_stage.at[c],
                    dst_ref=rs_recv.at[slot],
                    send_sem=rs_send_sems.at[c],
                    recv_sem=rs_recv_sems.at[slot],
                    device_id=(p,), device_id_type=pl.DeviceIdType.MESH,
                ).start()

        for slot in range(N_DEV - 1):
            pltpu.make_async_remote_copy(
                src_ref=rs_recv.at[slot], dst_ref=rs_recv.at[slot],
                send_sem=rs_send_sems.at[0],
                recv_sem=rs_recv_sems.at[slot],
                device_id=(0,), device_id_type=pl.DeviceIdType.MESH,
            ).wait_recv()
        red = out_ref[:, pl.ds(own * CHK, CHK), :]
        for slot in range(N_DEV - 1):
            red = red + rs_recv[slot].astype(jnp.float32)
        out_ref[:, pl.ds(own * CHK, CHK), :] = red
        ag_stage[:] = red.astype(jnp.bfloat16)

        for off in range(1, N_DEV):
            p = (my + off) % N_DEV
            pltpu.make_async_remote_copy(
                src_ref=ag_stage,
                dst_ref=ag_recv.at[N_DEV - 1 - off],
                send_sem=ag_send_sems.at[off - 1],
                recv_sem=ag_recv_sems.at[N_DEV - 1 - off],
                device_id=(p,), device_id_type=pl.DeviceIdType.MESH,
            ).start()
        for off in range(1, N_DEV):
            slot = off - 1
            pltpu.make_async_remote_copy(
                src_ref=ag_recv.at[slot], dst_ref=ag_recv.at[slot],
                send_sem=ag_send_sems.at[0],
                recv_sem=ag_recv_sems.at[slot],
                device_id=(0,), device_id_type=pl.DeviceIdType.MESH,
            ).wait_recv()
            src_chunk = (my + off + 1) % N_DEV
            out_ref[:, pl.ds(src_chunk * CHK, CHK), :] = (
                ag_recv[slot].astype(jnp.float32))

        @pl.when(is_src)
        def _():
            for hf in range(2):
                for d in range(1, N_DEV):
                    pltpu.make_async_remote_copy(
                        src_ref=kv_send.at[hf, d - 1],
                        dst_ref=kv_buf.at[:, :, pl.ds(hf * HALF, HALF), :],
                        send_sem=scat_send_sems.at[hf, d - 1],
                        recv_sem=scat_recv_sems.at[hf],
                        device_id=(d,), device_id_type=pl.DeviceIdType.MESH,
                    ).wait_send()
        for c in range(N_DEV):
            @pl.when(jnp.int32(c) != own)
            def _():
                pltpu.make_async_remote_copy(
                    src_ref=rs_stage.at[c], dst_ref=rs_recv.at[0],
                    send_sem=rs_send_sems.at[c],
                    recv_sem=rs_recv_sems.at[0],
                    device_id=(0,), device_id_type=pl.DeviceIdType.MESH,
                ).wait_send()
        for off in range(1, N_DEV):
            pltpu.make_async_remote_copy(
                src_ref=ag_stage, dst_ref=ag_recv.at[0],
                send_sem=ag_send_sems.at[off - 1],
                recv_sem=ag_recv_sems.at[0],
                device_id=(0,), device_id_type=pl.DeviceIdType.MESH,
            ).wait_send()

    out_shape = jax.ShapeDtypeStruct((B, SQ, DMODEL), jnp.float32)
    return pl.pallas_call(
        body,
        out_shape=out_shape,
        in_specs=[
            pl.BlockSpec(memory_space=pltpu.VMEM),
            pl.BlockSpec(memory_space=pltpu.VMEM),
            pl.BlockSpec(memory_space=pl.ANY),
            pl.BlockSpec(memory_space=pl.ANY),
            pl.BlockSpec(memory_space=pltpu.VMEM),
        ],
        out_specs=pl.BlockSpec(memory_space=pltpu.VMEM),
        scratch_shapes=[
            pltpu.VMEM((2, B, SKV, HQ_TOT * DH), jnp.bfloat16),
            pltpu.VMEM((2, N_DEV - 1, 2, B, HALF, HF), jnp.bfloat16),
            pltpu.VMEM((2, B, SKV, HF), jnp.bfloat16),
            pltpu.VMEM((N_DEV, B, CHK, DMODEL), jnp.bfloat16),
            pltpu.VMEM((N_DEV - 1, B, CHK, DMODEL), jnp.bfloat16),
            pltpu.VMEM((B, CHK, DMODEL), jnp.bfloat16),
            pltpu.VMEM((N_DEV - 1, B, CHK, DMODEL), jnp.bfloat16),
            pltpu.SemaphoreType.DMA((2, 2)),
            pltpu.SemaphoreType.DMA((2, N_DEV - 1)),
            pltpu.SemaphoreType.DMA((2,)),
            pltpu.SemaphoreType.DMA((N_DEV,)),
            pltpu.SemaphoreType.DMA((N_DEV - 1,)),
            pltpu.SemaphoreType.DMA((N_DEV - 1,)),
            pltpu.SemaphoreType.DMA((N_DEV - 1,)),
        ],
        compiler_params=pltpu.CompilerParams(
            collective_id=0,
            vmem_limit_bytes=120 * 1024 * 1024,
        ),
    )(xb, Wqb, K2, V2, Wob)


# device time: 99415 ns/iter; 3.5241x vs baseline; 1.0528x over previous
import os

import jax
import jax.numpy as jnp
from jax import lax
from jax.experimental import pallas as pl
from jax.experimental.pallas import tpu as pltpu

_SKIP_COMPUTE = os.environ.get("KSKIP_COMPUTE") == "1"
_SKIP_AR = os.environ.get("KSKIP_AR") == "1"

N_DEV = 4
B, SQ, DMODEL = 2, 512, 768
HQ_TOT, DH = 32, 64
H_LOC = HQ_TOT // N_DEV
HF = H_LOC * DH
SKV = 512
HALF = SKV // 2
BLK = 64
NEG = -1e9
CHK = SQ // N_DEV


def kernel(x, Wq, K_ext, V_ext, Wo):
    K2 = K_ext.reshape(B, SKV, HQ_TOT * DH).astype(jnp.bfloat16)
    V2 = V_ext.reshape(B, SKV, HQ_TOT * DH).astype(jnp.bfloat16)
    xb = x.astype(jnp.bfloat16)
    Wqb = Wq.astype(jnp.bfloat16)
    Wob = Wo.astype(jnp.bfloat16)

    def body(x_ref, wq_ref, k_ref, v_ref, wo_ref, out_ref,
             kvfull, kv_send, kv_buf, relay_buf,
             rs_stage, rs_recv, ag_stage, ag_recv,
             full_sems, scat_send_sems, scat_recv_sems, relay_recv_sems,
             fw_send_sems,
             rs_send_sems, rs_recv_sems, ag_send_sems, ag_recv_sems):
        my = lax.axis_index("i")
        is_src = my == 0
        own = (my + 1) % N_DEV

        bar = pltpu.get_barrier_semaphore()
        for off in range(1, N_DEV):
            peer = (my + off) % N_DEV
            pl.semaphore_signal(
                bar, inc=1,
                device_id=(peer,), device_id_type=pl.DeviceIdType.MESH,
            )

        @pl.when(is_src)
        def _():
            for hf in range(2):
                rows = pl.ds(hf * HALF, HALF)
                pltpu.make_async_copy(
                    k_ref.at[:, rows, :], kvfull.at[0, :, rows, :],
                    full_sems.at[hf, 0]).start()
                pltpu.make_async_copy(
                    v_ref.at[:, rows, :], kvfull.at[1, :, rows, :],
                    full_sems.at[hf, 1]).start()

        wo_bf = wo_ref[:]
        qs = []
        for b in range(B):
            q = jnp.dot(x_ref[b], wq_ref[:],
                        preferred_element_type=jnp.float32)
            qs.append(q.reshape(SQ, H_LOC, DH).astype(jnp.bfloat16))

        pl.semaphore_wait(bar, N_DEV - 1)

        @pl.when(is_src)
        def _():
            for hf in range(2):
                rows = pl.ds(hf * HALF, HALF)
                pltpu.make_async_copy(
                    k_ref.at[:, rows, :], kvfull.at[0, :, rows, :],
                    full_sems.at[hf, 0]).wait()
                pltpu.make_async_copy(
                    v_ref.at[:, rows, :], kvfull.at[1, :, rows, :],
                    full_sems.at[hf, 1]).wait()
                kvh = kvfull[:, :, hf * HALF:(hf + 1) * HALF, :]
                for d in range(1, N_DEV):
                    kv_send[hf, d - 1] = kvh[:, :, :, d * HF:(d + 1) * HF]
                sends = [
                    (kv_send.at[hf, 0], kv_buf.at[:, :, rows, :],
                     scat_recv_sems, 1),
                    (kv_send.at[hf, 1, 0], relay_buf.at[hf],
                     relay_recv_sems, 1),
                    (kv_send.at[hf, 2], kv_buf.at[:, :, rows, :],
                     scat_recv_sems, 3),
                    (kv_send.at[hf, 1, 1], relay_buf.at[hf],
                     relay_recv_sems, 3),
                ]
                for i, (src, dst, rsem, dev) in enumerate(sends):
                    pltpu.make_async_remote_copy(
                        src_ref=src, dst_ref=dst,
                        send_sem=scat_send_sems.at[hf, i],
                        recv_sem=rsem.at[hf],
                        device_id=(dev,),
                        device_id_type=pl.DeviceIdType.MESH,
                    ).start()
                kv_buf[:, :, rows, :] = kvh[:, :, :, 0:HF]

        def relay_wait_fwd(hf, piece):
            rows = pl.ds(hf * HALF, HALF)
            pltpu.make_async_remote_copy(
                src_ref=relay_buf.at[hf], dst_ref=relay_buf.at[hf],
                send_sem=fw_send_sems.at[hf],
                recv_sem=relay_recv_sems.at[hf],
                device_id=(0,), device_id_type=pl.DeviceIdType.MESH,
            ).wait_recv()
            pltpu.make_async_remote_copy(
                src_ref=relay_buf.at[hf],
                dst_ref=kv_buf.at[piece, :, rows, :],
                send_sem=fw_send_sems.at[hf],
                recv_sem=(scat_recv_sems if piece == 0
                          else relay_recv_sems).at[hf],
                device_id=(2,), device_id_type=pl.DeviceIdType.MESH,
            ).start()

        def main_wait(hf):
            rows = pl.ds(hf * HALF, HALF)
            pltpu.make_async_remote_copy(
                src_ref=kv_buf.at[:, :, rows, :],
                dst_ref=kv_buf.at[:, :, rows, :],
                send_sem=fw_send_sems.at[hf],
                recv_sem=scat_recv_sems.at[hf],
                device_id=(0,), device_id_type=pl.DeviceIdType.MESH,
            ).wait_recv()

        def mid_wait(hf):
            rows = pl.ds(hf * HALF, HALF)
            for piece, rsem in ((0, scat_recv_sems), (1, relay_recv_sems)):
                pltpu.make_async_remote_copy(
                    src_ref=kv_buf.at[piece, :, rows, :],
                    dst_ref=kv_buf.at[piece, :, rows, :],
                    send_sem=fw_send_sems.at[hf],
                    recv_sem=rsem.at[hf],
                    device_id=(0,), device_id_type=pl.DeviceIdType.MESH,
                ).wait_recv()

        qb_base = lax.broadcasted_iota(jnp.int32, (CHK, SKV), 0)
        kb_full = lax.broadcasted_iota(jnp.int32, (CHK, SKV), 1) // BLK

        for c in range(N_DEV):
            if c == 0:
                @pl.when(my == 1)
                def _():
                    relay_wait_fwd(0, 0)
                    main_wait(0)
                @pl.when(my == 3)
                def _():
                    relay_wait_fwd(0, 1)
                    main_wait(0)
                @pl.when(my == 2)
                def _():
                    mid_wait(0)
            if c == 1:
                @pl.when(my == 1)
                def _():
                    relay_wait_fwd(1, 0)
                @pl.when(my == 3)
                def _():
                    relay_wait_fwd(1, 1)
            if c == 2:
                @pl.when((my == 1) | (my == 3))
                def _():
                    main_wait(1)
                @pl.when(my == 2)
                def _():
                    mid_wait(1)
            kl = CHK * (c + 1)
            mask = kb_full[:, :kl] <= (qb_base[:, :kl] + c * CHK) // BLK
            if _SKIP_COMPUTE:
                for b in range(B):
                    z = jnp.zeros((CHK, DMODEL), jnp.float32)
                    out_ref[b, c * CHK:(c + 1) * CHK, :] = z
                    rs_stage[c, b] = z.astype(jnp.bfloat16)
            for b in ([] if _SKIP_COMPUTE else range(B)):
                ctx_parts = []
                for h in range(H_LOC):
                    kh = kv_buf[0, b, 0:kl, h * DH:(h + 1) * DH]
                    vh = kv_buf[1, b, 0:kl, h * DH:(h + 1) * DH]
                    qch = qs[b][c * CHK:(c + 1) * CHK, h, :]
                    s = lax.dot_general(
                        qch, kh, (((1,), (1,)), ((), ())),
                        preferred_element_type=jnp.float32) * 0.125
                    s = jnp.where(mask, s, NEG)
                    m = jnp.max(s, axis=-1, keepdims=True)
                    w = jnp.exp(s - m)
                    w = w / jnp.sum(w, axis=-1, keepdims=True)
                    ctx_parts.append(
                        jnp.dot(w.astype(jnp.bfloat16), vh,
                                preferred_element_type=jnp.float32))
                ctx = jnp.concatenate(ctx_parts, axis=-1)
                part = jnp.dot(ctx.astype(jnp.bfloat16), wo_bf,
                               preferred_element_type=jnp.float32)
                out_ref[b, c * CHK:(c + 1) * CHK, :] = part
                rs_stage[c, b] = part.astype(jnp.bfloat16)
            if _SKIP_AR:
                continue
            p = (c - 1) % N_DEV
            @pl.when(jnp.int32(c) != own)
            def _():
                slot = (my - p) % N_DEV - 1
                pltpu.make_async_remote_copy(
                    src_ref=rs_stage.at[c],
                    dst_ref=rs_recv.at[slot],
                    send_sem=rs_send_sems.at[c],
                    recv_sem=rs_recv_sems.at[slot],
                    device_id=(p,), device_id_type=pl.DeviceIdType.MESH,
                ).start()

        for slot in ([] if _SKIP_AR else range(N_DEV - 1)):
            pltpu.make_async_remote_copy(
                src_ref=rs_recv.at[slot], dst_ref=rs_recv.at[slot],
                send_sem=rs_send_sems.at[0],
                recv_sem=rs_recv_sems.at[slot],
                device_id=(0,), device_id_type=pl.DeviceIdType.MESH,
            ).wait_recv()
        if not _SKIP_AR:
            red = out_ref[:, pl.ds(own * CHK, CHK), :]
            for slot in range(N_DEV - 1):
                red = red + rs_recv[slot].astype(jnp.float32)
            out_ref[:, pl.ds(own * CHK, CHK), :] = red
            ag_stage[:] = red.astype(jnp.bfloat16)

        for off in ([] if _SKIP_AR else range(1, N_DEV)):
            p = (my + off) % N_DEV
            pltpu.make_async_remote_copy(
                src_ref=ag_stage,
                dst_ref=ag_recv.at[N_DEV - 1 - off],
                send_sem=ag_send_sems.at[off - 1],
                recv_sem=ag_recv_sems.at[N_DEV - 1 - off],
                device_id=(p,), device_id_type=pl.DeviceIdType.MESH,
            ).start()
        for off in ([] if _SKIP_AR else range(1, N_DEV)):
            slot = off - 1
            pltpu.make_async_remote_copy(
                src_ref=ag_recv.at[slot], dst_ref=ag_recv.at[slot],
                send_sem=ag_send_sems.at[0],
                recv_sem=ag_recv_sems.at[slot],
                device_id=(0,), device_id_type=pl.DeviceIdType.MESH,
            ).wait_recv()
            src_chunk = (my + off + 1) % N_DEV
            out_ref[:, pl.ds(src_chunk * CHK, CHK), :] = (
                ag_recv[slot].astype(jnp.float32))

        @pl.when(is_src)
        def _():
            for hf in range(2):
                rows = pl.ds(hf * HALF, HALF)
                drains = [
                    (kv_send.at[hf, 0], kv_buf.at[:, :, rows, :], 1),
                    (kv_send.at[hf, 1, 0], relay_buf.at[hf], 1),
                    (kv_send.at[hf, 2], kv_buf.at[:, :, rows, :], 3),
                    (kv_send.at[hf, 1, 1], relay_buf.at[hf], 3),
                ]
                for i, (src, dst, dev) in enumerate(drains):
                    pltpu.make_async_remote_copy(
                        src_ref=src, dst_ref=dst,
                        send_sem=scat_send_sems.at[hf, i],
                        recv_sem=scat_recv_sems.at[hf],
                        device_id=(dev,),
                        device_id_type=pl.DeviceIdType.MESH,
                    ).wait_send()

        @pl.when((my == 1) | (my == 3))
        def _():
            for hf in range(2):
                pltpu.make_async_remote_copy(
                    src_ref=relay_buf.at[hf],
                    dst_ref=kv_buf.at[0, :, pl.ds(hf * HALF, HALF), :],
                    send_sem=fw_send_sems.at[hf],
                    recv_sem=relay_recv_sems.at[hf],
                    device_id=(2,), device_id_type=pl.DeviceIdType.MESH,
                ).wait_send()
        for c in ([] if _SKIP_AR else range(N_DEV)):
            @pl.when(jnp.int32(c) != own)
            def _():
                pltpu.make_async_remote_copy(
                    src_ref=rs_stage.at[c], dst_ref=rs_recv.at[0],
                    send_sem=rs_send_sems.at[c],
                    recv_sem=rs_recv_sems.at[0],
                    device_id=(0,), device_id_type=pl.DeviceIdType.MESH,
                ).wait_send()
        for off in ([] if _SKIP_AR else range(1, N_DEV)):
            pltpu.make_async_remote_copy(
                src_ref=ag_stage, dst_ref=ag_recv.at[0],
                send_sem=ag_send_sems.at[off - 1],
                recv_sem=ag_recv_sems.at[0],
                device_id=(0,), device_id_type=pl.DeviceIdType.MESH,
            ).wait_send()

    out_shape = jax.ShapeDtypeStruct((B, SQ, DMODEL), jnp.float32)
    return pl.pallas_call(
        body,
        out_shape=out_shape,
        in_specs=[
            pl.BlockSpec(memory_space=pltpu.VMEM),
            pl.BlockSpec(memory_space=pltpu.VMEM),
            pl.BlockSpec(memory_space=pl.ANY),
            pl.BlockSpec(memory_space=pl.ANY),
            pl.BlockSpec(memory_space=pltpu.VMEM),
        ],
        out_specs=pl.BlockSpec(memory_space=pltpu.VMEM),
        scratch_shapes=[
            pltpu.VMEM((2, B, SKV, HQ_TOT * DH), jnp.bfloat16),
            pltpu.VMEM((2, N_DEV - 1, 2, B, HALF, HF), jnp.bfloat16),
            pltpu.VMEM((2, B, SKV, HF), jnp.bfloat16),
            pltpu.VMEM((2, B, HALF, HF), jnp.bfloat16),
            pltpu.VMEM((N_DEV, B, CHK, DMODEL), jnp.bfloat16),
            pltpu.VMEM((N_DEV - 1, B, CHK, DMODEL), jnp.bfloat16),
            pltpu.VMEM((B, CHK, DMODEL), jnp.bfloat16),
            pltpu.VMEM((N_DEV - 1, B, CHK, DMODEL), jnp.bfloat16),
            pltpu.SemaphoreType.DMA((2, 2)),
            pltpu.SemaphoreType.DMA((2, 4)),
            pltpu.SemaphoreType.DMA((2,)),
            pltpu.SemaphoreType.DMA((2,)),
            pltpu.SemaphoreType.DMA((2,)),
            pltpu.SemaphoreType.DMA((N_DEV,)),
            pltpu.SemaphoreType.DMA((N_DEV - 1,)),
            pltpu.SemaphoreType.DMA((N_DEV - 1,)),
            pltpu.SemaphoreType.DMA((N_DEV - 1,)),
        ],
        compiler_params=pltpu.CompilerParams(
            collective_id=0,
            vmem_limit_bytes=120 * 1024 * 1024,
        ),
    )(xb, Wqb, K2, V2, Wob)


# device time: 89955 ns/iter; 3.8947x vs baseline; 1.1052x over previous
import os

import jax
import jax.numpy as jnp
from jax import lax
from jax.experimental import pallas as pl
from jax.experimental.pallas import tpu as pltpu

_SKIP_COMPUTE = os.environ.get("KSKIP_COMPUTE") == "1"
_SKIP_AR = os.environ.get("KSKIP_AR") == "1"

N_DEV = 4
B, SQ, DMODEL = 2, 512, 768
HQ_TOT, DH = 32, 64
H_LOC = HQ_TOT // N_DEV
HF = H_LOC * DH
SKV = 512
NQ = 4
QT = SKV // NQ
BLK = 64
NEG = -1e9
CHK = SQ // N_DEV


def kernel(x, Wq, K_ext, V_ext, Wo):
    K2 = K_ext.reshape(B, SKV, HQ_TOT * DH).astype(jnp.bfloat16)
    V2 = V_ext.reshape(B, SKV, HQ_TOT * DH).astype(jnp.bfloat16)
    xb = x.astype(jnp.bfloat16)
    Wqb = Wq.astype(jnp.bfloat16)
    Wob = Wo.astype(jnp.bfloat16)

    def body(x_ref, wq_ref, k_ref, v_ref, wo_ref, out_ref,
             kvfull, kv_send, kv_buf, relay_buf,
             rs_stage, rs_recv, ag_stage, ag_recv,
             full_sems, scat_send_sems, scat_recv_sems, relay_recv_sems,
             fw_send_sems,
             rs_send_sems, rs_recv_sems, ag_send_sems, ag_recv_sems):
        my = lax.axis_index("i")
        is_src = my == 0
        own = (my + 1) % N_DEV

        bar = pltpu.get_barrier_semaphore()
        for off in range(1, N_DEV):
            peer = (my + off) % N_DEV
            pl.semaphore_signal(
                bar, inc=1,
                device_id=(peer,), device_id_type=pl.DeviceIdType.MESH,
            )

        @pl.when(is_src)
        def _():
            for qt in range(NQ):
                rows = pl.ds(qt * QT, QT)
                pltpu.make_async_copy(
                    k_ref.at[:, rows, :], kvfull.at[0, :, rows, :],
                    full_sems.at[qt, 0]).start()
                pltpu.make_async_copy(
                    v_ref.at[:, rows, :], kvfull.at[1, :, rows, :],
                    full_sems.at[qt, 1]).start()

        wo_bf = wo_ref[:]
        qs = []
        for b in range(B):
            q = jnp.dot(x_ref[b], wq_ref[:],
                        preferred_element_type=jnp.float32)
            qs.append(q.reshape(SQ, H_LOC, DH).astype(jnp.bfloat16))

        pl.semaphore_wait(bar, N_DEV - 1)

        @pl.when(is_src)
        def _():
            for qt in range(NQ):
                rows = pl.ds(qt * QT, QT)
                pltpu.make_async_copy(
                    k_ref.at[:, rows, :], kvfull.at[0, :, rows, :],
                    full_sems.at[qt, 0]).wait()
                pltpu.make_async_copy(
                    v_ref.at[:, rows, :], kvfull.at[1, :, rows, :],
                    full_sems.at[qt, 1]).wait()
                kvh = kvfull[:, :, qt * QT:(qt + 1) * QT, :]
                for d in range(1, N_DEV):
                    kv_send[qt, d - 1] = kvh[:, :, :, d * HF:(d + 1) * HF]
                sends = [
                    (kv_send.at[qt, 0], kv_buf.at[:, :, rows, :],
                     scat_recv_sems, 1),
                    (kv_send.at[qt, 1, 0], relay_buf.at[qt],
                     relay_recv_sems, 1),
                    (kv_send.at[qt, 2], kv_buf.at[:, :, rows, :],
                     scat_recv_sems, 3),
                    (kv_send.at[qt, 1, 1], relay_buf.at[qt],
                     relay_recv_sems, 3),
                ]
                for i, (src, dst, rsem, dev) in enumerate(sends):
                    pltpu.make_async_remote_copy(
                        src_ref=src, dst_ref=dst,
                        send_sem=scat_send_sems.at[qt, i],
                        recv_sem=rsem.at[qt],
                        device_id=(dev,),
                        device_id_type=pl.DeviceIdType.MESH,
                    ).start()
                kv_buf[:, :, rows, :] = kvh[:, :, :, 0:HF]

        def relay_wait_fwd(qt, piece):
            rows = pl.ds(qt * QT, QT)
            pltpu.make_async_remote_copy(
                src_ref=relay_buf.at[qt], dst_ref=relay_buf.at[qt],
                send_sem=fw_send_sems.at[qt],
                recv_sem=relay_recv_sems.at[qt],
                device_id=(0,), device_id_type=pl.DeviceIdType.MESH,
            ).wait_recv()
            pltpu.make_async_remote_copy(
                src_ref=relay_buf.at[qt],
                dst_ref=kv_buf.at[piece, :, rows, :],
                send_sem=fw_send_sems.at[qt],
                recv_sem=(scat_recv_sems if piece == 0
                          else relay_recv_sems).at[qt],
                device_id=(2,), device_id_type=pl.DeviceIdType.MESH,
            ).start()

        def main_wait(qt):
            rows = pl.ds(qt * QT, QT)
            pltpu.make_async_remote_copy(
                src_ref=kv_buf.at[:, :, rows, :],
                dst_ref=kv_buf.at[:, :, rows, :],
                send_sem=fw_send_sems.at[qt],
                recv_sem=scat_recv_sems.at[qt],
                device_id=(0,), device_id_type=pl.DeviceIdType.MESH,
            ).wait_recv()

        def mid_wait(qt):
            rows = pl.ds(qt * QT, QT)
            for piece, rsem in ((0, scat_recv_sems), (1, relay_recv_sems)):
                pltpu.make_async_remote_copy(
                    src_ref=kv_buf.at[piece, :, rows, :],
                    dst_ref=kv_buf.at[piece, :, rows, :],
                    send_sem=fw_send_sems.at[qt],
                    recv_sem=rsem.at[qt],
                    device_id=(0,), device_id_type=pl.DeviceIdType.MESH,
                ).wait_recv()

        qb_base = lax.broadcasted_iota(jnp.int32, (CHK, SKV), 0)
        kb_full = lax.broadcasted_iota(jnp.int32, (CHK, SKV), 1) // BLK

        for c in range(N_DEV):
            @pl.when(my == 1)
            def _():
                relay_wait_fwd(c, 0)
                main_wait(c)
            @pl.when(my == 3)
            def _():
                relay_wait_fwd(c, 1)
                main_wait(c)
            @pl.when(my == 2)
            def _():
                mid_wait(c)
            kl = CHK * (c + 1)
            mask = kb_full[:, :kl] <= (qb_base[:, :kl] + c * CHK) // BLK
            if _SKIP_COMPUTE:
                for b in range(B):
                    z = jnp.zeros((CHK, DMODEL), jnp.float32)
                    out_ref[b, c * CHK:(c + 1) * CHK, :] = z
                    rs_stage[c, b] = z.astype(jnp.bfloat16)
            for b in ([] if _SKIP_COMPUTE else range(B)):
                ctx_parts = []
                for h in range(H_LOC):
                    kh = kv_buf[0, b, 0:kl, h * DH:(h + 1) * DH]
                    vh = kv_buf[1, b, 0:kl, h * DH:(h + 1) * DH]
                    qch = qs[b][c * CHK:(c + 1) * CHK, h, :]
                    s = lax.dot_general(
                        qch, kh, (((1,), (1,)), ((), ())),
                        preferred_element_type=jnp.float32) * 0.125
                    s = jnp.where(mask, s, NEG)
                    m = jnp.max(s, axis=-1, keepdims=True)
                    w = jnp.exp(s - m)
                    w = w / jnp.sum(w, axis=-1, keepdims=True)
                    ctx_parts.append(
                        jnp.dot(w.astype(jnp.bfloat16), vh,
                                preferred_element_type=jnp.float32))
                ctx = jnp.concatenate(ctx_parts, axis=-1)
                part = jnp.dot(ctx.astype(jnp.bfloat16), wo_bf,
                               preferred_element_type=jnp.float32)
                out_ref[b, c * CHK:(c + 1) * CHK, :] = part
                rs_stage[c, b] = part.astype(jnp.bfloat16)
            if _SKIP_AR:
                continue
            p = (c - 1) % N_DEV
            @pl.when(jnp.int32(c) != own)
            def _():
                slot = (my - p) % N_DEV - 1
                pltpu.make_async_remote_copy(
                    src_ref=rs_stage.at[c],
                    dst_ref=rs_recv.at[slot],
                    send_sem=rs_send_sems.at[c],
                    recv_sem=rs_recv_sems.at[slot],
                    device_id=(p,), device_id_type=pl.DeviceIdType.MESH,
                ).start()

        for slot in ([] if _SKIP_AR else range(N_DEV - 1)):
            pltpu.make_async_remote_copy(
                src_ref=rs_recv.at[slot], dst_ref=rs_recv.at[slot],
                send_sem=rs_send_sems.at[0],
                recv_sem=rs_recv_sems.at[slot],
                device_id=(0,), device_id_type=pl.DeviceIdType.MESH,
            ).wait_recv()
        if not _SKIP_AR:
            red = out_ref[:, pl.ds(own * CHK, CHK), :]
            for slot in range(N_DEV - 1):
                red = red + rs_recv[slot].astype(jnp.float32)
            out_ref[:, pl.ds(own * CHK, CHK), :] = red
            ag_stage[:] = red.astype(jnp.bfloat16)

        for off in ([] if _SKIP_AR else range(1, N_DEV)):
            p = (my + off) % N_DEV
            pltpu.make_async_remote_copy(
                src_ref=ag_stage,
                dst_ref=ag_recv.at[N_DEV - 1 - off],
                send_sem=ag_send_sems.at[off - 1],
                recv_sem=ag_recv_sems.at[N_DEV - 1 - off],
                device_id=(p,), device_id_type=pl.DeviceIdType.MESH,
            ).start()
        for off in ([] if _SKIP_AR else range(1, N_DEV)):
            slot = off - 1
            pltpu.make_async_remote_copy(
                src_ref=ag_recv.at[slot], dst_ref=ag_recv.at[slot],
                send_sem=ag_send_sems.at[0],
                recv_sem=ag_recv_sems.at[slot],
                device_id=(0,), device_id_type=pl.DeviceIdType.MESH,
            ).wait_recv()
            src_chunk = (my + off + 1) % N_DEV
            out_ref[:, pl.ds(src_chunk * CHK, CHK), :] = (
                ag_recv[slot].astype(jnp.float32))

        @pl.when(is_src)
        def _():
            for qt in range(NQ):
                rows = pl.ds(qt * QT, QT)
                drains = [
                    (kv_send.at[qt, 0], kv_buf.at[:, :, rows, :], 1),
                    (kv_send.at[qt, 1, 0], relay_buf.at[qt], 1),
                    (kv_send.at[qt, 2], kv_buf.at[:, :, rows, :], 3),
                    (kv_send.at[qt, 1, 1], relay_buf.at[qt], 3),
                ]
                for i, (src, dst, dev) in enumerate(drains):
                    pltpu.make_async_remote_copy(
                        src_ref=src, dst_ref=dst,
                        send_sem=scat_send_sems.at[qt, i],
                        recv_sem=scat_recv_sems.at[qt],
                        device_id=(dev,),
                        device_id_type=pl.DeviceIdType.MESH,
                    ).wait_send()

        @pl.when((my == 1) | (my == 3))
        def _():
            for qt in range(NQ):
                pltpu.make_async_remote_copy(
                    src_ref=relay_buf.at[qt],
                    dst_ref=kv_buf.at[0, :, pl.ds(qt * QT, QT), :],
                    send_sem=fw_send_sems.at[qt],
                    recv_sem=relay_recv_sems.at[qt],
                    device_id=(2,), device_id_type=pl.DeviceIdType.MESH,
                ).wait_send()
        for c in ([] if _SKIP_AR else range(N_DEV)):
            @pl.when(jnp.int32(c) != own)
            def _():
                pltpu.make_async_remote_copy(
                    src_ref=rs_stage.at[c], dst_ref=rs_recv.at[0],
                    send_sem=rs_send_sems.at[c],
                    recv_sem=rs_recv_sems.at[0],
                    device_id=(0,), device_id_type=pl.DeviceIdType.MESH,
                ).wait_send()
        for off in ([] if _SKIP_AR else range(1, N_DEV)):
            pltpu.make_async_remote_copy(
                src_ref=ag_stage, dst_ref=ag_recv.at[0],
                send_sem=ag_send_sems.at[off - 1],
                recv_sem=ag_recv_sems.at[0],
                device_id=(0,), device_id_type=pl.DeviceIdType.MESH,
            ).wait_send()

    out_shape = jax.ShapeDtypeStruct((B, SQ, DMODEL), jnp.float32)
    return pl.pallas_call(
        body,
        out_shape=out_shape,
        in_specs=[
            pl.BlockSpec(memory_space=pltpu.VMEM),
            pl.BlockSpec(memory_space=pltpu.VMEM),
            pl.BlockSpec(memory_space=pl.ANY),
            pl.BlockSpec(memory_space=pl.ANY),
            pl.BlockSpec(memory_space=pltpu.VMEM),
        ],
        out_specs=pl.BlockSpec(memory_space=pltpu.VMEM),
        scratch_shapes=[
            pltpu.VMEM((2, B, SKV, HQ_TOT * DH), jnp.bfloat16),
            pltpu.VMEM((NQ, N_DEV - 1, 2, B, QT, HF), jnp.bfloat16),
            pltpu.VMEM((2, B, SKV, HF), jnp.bfloat16),
            pltpu.VMEM((NQ, B, QT, HF), jnp.bfloat16),
            pltpu.VMEM((N_DEV, B, CHK, DMODEL), jnp.bfloat16),
            pltpu.VMEM((N_DEV - 1, B, CHK, DMODEL), jnp.bfloat16),
            pltpu.VMEM((B, CHK, DMODEL), jnp.bfloat16),
            pltpu.VMEM((N_DEV - 1, B, CHK, DMODEL), jnp.bfloat16),
            pltpu.SemaphoreType.DMA((NQ, 2)),
            pltpu.SemaphoreType.DMA((NQ, 4)),
            pltpu.SemaphoreType.DMA((NQ,)),
            pltpu.SemaphoreType.DMA((NQ,)),
            pltpu.SemaphoreType.DMA((NQ,)),
            pltpu.SemaphoreType.DMA((N_DEV,)),
            pltpu.SemaphoreType.DMA((N_DEV - 1,)),
            pltpu.SemaphoreType.DMA((N_DEV - 1,)),
            pltpu.SemaphoreType.DMA((N_DEV - 1,)),
        ],
        compiler_params=pltpu.CompilerParams(
            collective_id=0,
            vmem_limit_bytes=120 * 1024 * 1024,
        ),
    )(xb, Wqb, K2, V2, Wob)
